# baseline (speedup 1.0000x reference)
import functools

import jax
import jax.numpy as jnp
from jax import lax
from jax.experimental import pallas as pl
from jax.experimental.pallas import tpu as pltpu
from jax.experimental.pallas import tpu_sc as plsc

N = 10000
D = 128
E = 320000

NC = 2
NS = 16
NW = NC * NS

CH = 64
EPW = 10240
E2 = NW * EPW
KCH = EPW // CH
NSL = 4

NPAD = 10112
RPT = NPAD // NS
HPAD = 10240
BPT = HPAD // NS
BLK = 16

_mesh = plsc.VectorSubcoreMesh(
    core_axis_name="c", subcore_axis_name="s", num_cores=NC, num_subcores=NS)


@functools.partial(
    pl.kernel,
    out_type=(jax.ShapeDtypeStruct((NC, 1, HPAD), jnp.float32),
              jax.ShapeDtypeStruct((NC, 1, HPAD), jnp.float32)),
    mesh=_mesh,
    compiler_params=pltpu.CompilerParams(needs_layout_passes=False),
    scratch_types=[
        pltpu.VMEM_SHARED((NS, HPAD), jnp.float32),
        pltpu.VMEM((HPAD,), jnp.float32),
        pltpu.VMEM((HPAD,), jnp.float32),
        pltpu.VMEM((EPW,), jnp.int32),
        pltpu.VMEM((EPW,), jnp.int32),
        pltpu.VMEM((BPT,), jnp.float32),
        pltpu.VMEM((BPT,), jnp.float32),
    ],
)
def _hist_k(src_hbm, dst_hbm, osrc_hbm, odst_hbm, stag, hist_s, hist_d,
            sidx_all, didx_all, rbuf, acc):
    c = lax.axis_index("c")
    s = lax.axis_index("s")
    w = c * NS + s

    pltpu.sync_copy(src_hbm.at[pl.ds(w * EPW, EPW)], sidx_all)
    pltpu.sync_copy(dst_hbm.at[pl.ds(w * EPW, EPW)], didx_all)

    def zfill(i, _):
        hist_s[pl.ds(i * 16, 16)] = jnp.zeros((16,), jnp.float32)
        hist_d[pl.ds(i * 16, 16)] = jnp.zeros((16,), jnp.float32)
        return 0
    lax.fori_loop(0, HPAD // 16, zfill, 0)

    def body(i, _):
        iv = sidx_all[pl.ds(i * 16, 16)]
        cnt, last = plsc.scan_count(iv)
        plsc.addupdate_scatter(hist_s, [iv], cnt.astype(jnp.float32),
                               mask=last)
        iv2 = didx_all[pl.ds(i * 16, 16)]
        cnt2, last2 = plsc.scan_count(iv2)
        plsc.addupdate_scatter(hist_d, [iv2], cnt2.astype(jnp.float32),
                               mask=last2)
        return 0
    lax.fori_loop(0, EPW // 16, body, 0)

    for hist_ref, out_ref in ((hist_s, osrc_hbm), (hist_d, odst_hbm)):
        pltpu.sync_copy(hist_ref, stag.at[s])
        plsc.subcore_barrier()
        pltpu.sync_copy(stag.at[0, pl.ds(s * BPT, BPT)], acc)

        def rsum(r, _):
            pltpu.sync_copy(stag.at[r, pl.ds(s * BPT, BPT)], rbuf)

            def vadd(v, _):
                acc[pl.ds(v * 16, 16)] = (acc[pl.ds(v * 16, 16)]
                                          + rbuf[pl.ds(v * 16, 16)])
                return 0
            lax.fori_loop(0, BPT // 16, vadd, 0)
            return 0
        lax.fori_loop(1, NS, rsum, 0)
        pltpu.sync_copy(acc, out_ref.at[c, 0, pl.ds(s * BPT, BPT)])
        plsc.subcore_barrier()


@functools.partial(
    pl.kernel,
    out_type=jax.ShapeDtypeStruct((NC, NPAD, D), jnp.float32),
    mesh=_mesh,
    scratch_types=[
        pltpu.VMEM_SHARED((NPAD, D), jnp.float32),
        pltpu.VMEM((BLK, CH), jnp.int32),
        pltpu.VMEM((BLK, CH), jnp.int32),
        pltpu.VMEM((NSL, CH, D), jnp.float32),
        pltpu.SemaphoreType.DMA,
        pltpu.SemaphoreType.DMA,
        pltpu.SemaphoreType.DMA,
        pltpu.SemaphoreType.DMA,
        pltpu.SemaphoreType.DMA,
        pltpu.SemaphoreType.DMA,
        pltpu.SemaphoreType.DMA,
        pltpu.SemaphoreType.DMA,
    ],
)
def _agg_k(h_hbm, src2_hbm, dst2_hbm, z_hbm, out_hbm, agg_sh, sblk, dblk,
           rows, g0, g1, g2, g3, s0, s1, s2, s3):
    c = lax.axis_index("c")
    s = lax.axis_index("s")
    w = c * NS + s

    pltpu.sync_copy(z_hbm, agg_sh.at[pl.ds(s * RPT, RPT)])
    plsc.subcore_barrier()

    gsem = (g0, g1, g2, g3)
    ssem = (s0, s1, s2, s3)

    def gather(j, slot):
        pltpu.async_copy(h_hbm.at[sblk.at[j]], rows.at[slot], gsem[slot])

    def wait_gather(j, slot):
        pltpu.make_async_copy(h_hbm.at[sblk.at[j]], rows.at[slot],
                              gsem[slot]).wait()

    def scatter(j, slot):
        pltpu.sync_copy(rows.at[slot], agg_sh.at[dblk.at[j]], add=True)

    def blk_body(b, _):
        row0 = w * KCH + b * BLK
        pltpu.sync_copy(src2_hbm.at[pl.ds(row0, BLK), :], sblk)
        pltpu.sync_copy(dst2_hbm.at[pl.ds(row0, BLK), :], dblk)
        for j in range(NSL - 1):
            gather(j, j % NSL)
        for j in range(BLK):
            slot = j % NSL
            wait_gather(j, slot)
            scatter(j, slot)
            nx = j + NSL - 1
            if nx < BLK:
                gather(nx, nx % NSL)
        return 0
    lax.fori_loop(0, KCH // BLK, blk_body, 0)

    plsc.subcore_barrier()
    pltpu.sync_copy(agg_sh.at[pl.ds(s * RPT, RPT)],
                    out_hbm.at[c, pl.ds(s * RPT, RPT), :])


def _mm_body(feat_ref, w_ref, p_ref):
    p_ref[...] = jnp.dot(feat_ref[...], w_ref[...],
                         preferred_element_type=jnp.float32)


def _prep_body(hsrc_ref, p_ref, h_ref):
    hs = hsrc_ref[0, 0, :] + hsrc_ref[1, 0, :]
    outd = jnp.maximum(hs[0:N], 1.0)
    nrm = lax.rsqrt(outd)
    h_ref[0:N, :] = p_ref[...] * nrm[:, None]
    h_ref[N:NPAD, :] = jnp.zeros((NPAD - N, D), jnp.float32)


def _final_body(agg_ref, hdst_ref, b_ref, out_ref):
    acc = agg_ref[0, 0:N, :] + agg_ref[1, 0:N, :]
    hd = hdst_ref[0, 0, :] + hdst_ref[1, 0, :]
    ind = jnp.maximum(hd[0:N], 1.0)
    out_ref[...] = acc * lax.rsqrt(ind)[:, None] + b_ref[0, :][None, :]


def kernel(feat, edge_index, W, b):
    pad = (jnp.arange(E2 - E, dtype=jnp.int32) % (NPAD - N)) + N
    srcp = jnp.concatenate([edge_index[0], pad])
    dstp = jnp.concatenate([edge_index[1], pad])

    hsrc, hdst = _hist_k(srcp, dstp)

    p = pl.pallas_call(
        _mm_body,
        out_shape=jax.ShapeDtypeStruct((N, D), jnp.float32),
    )(feat, W)

    h = pl.pallas_call(
        _prep_body,
        out_shape=jax.ShapeDtypeStruct((NPAD, D), jnp.float32),
    )(hsrc, p)

    zeros = jnp.zeros((RPT, D), jnp.float32)
    agg = _agg_k(h, srcp.reshape(E2 // CH, CH),
                 dstp.reshape(E2 // CH, CH), zeros)

    out = pl.pallas_call(
        _final_body,
        out_shape=jax.ShapeDtypeStruct((N, D), jnp.float32),
    )(agg, hdst, b.reshape(1, D))
    return out


# speedup vs baseline: 1.0652x; 1.0652x over previous
import functools

import jax
import jax.numpy as jnp
from jax import lax
from jax.experimental import pallas as pl
from jax.experimental.pallas import tpu as pltpu
from jax.experimental.pallas import tpu_sc as plsc

N = 10000
D = 128
E = 320000

NC = 2
NS = 16
NW = NC * NS

CH = 128
EPW = 10240
E2 = NW * EPW
KCH = EPW // CH

NPAD = 10112
RPT = NPAD // NS
HPAD = 10240
BPT = HPAD // NS
BLK = 16

_mesh = plsc.VectorSubcoreMesh(
    core_axis_name="c", subcore_axis_name="s", num_cores=NC, num_subcores=NS)


@functools.partial(
    pl.kernel,
    out_type=(jax.ShapeDtypeStruct((NC, 1, HPAD), jnp.float32),
              jax.ShapeDtypeStruct((NC, 1, HPAD), jnp.float32)),
    mesh=_mesh,
    compiler_params=pltpu.CompilerParams(needs_layout_passes=False),
    scratch_types=[
        pltpu.VMEM_SHARED((NS, HPAD), jnp.float32),
        pltpu.VMEM((HPAD,), jnp.float32),
        pltpu.VMEM((HPAD,), jnp.float32),
        pltpu.VMEM((EPW,), jnp.int32),
        pltpu.VMEM((EPW,), jnp.int32),
        pltpu.VMEM((BPT,), jnp.float32),
        pltpu.VMEM((BPT,), jnp.float32),
    ],
)
def _hist_k(src_hbm, dst_hbm, osrc_hbm, odst_hbm, stag, hist_s, hist_d,
            sidx_all, didx_all, rbuf, acc):
    c = lax.axis_index("c")
    s = lax.axis_index("s")
    w = c * NS + s

    pltpu.sync_copy(src_hbm.at[pl.ds(w * EPW, EPW)], sidx_all)
    pltpu.sync_copy(dst_hbm.at[pl.ds(w * EPW, EPW)], didx_all)

    def zfill(i, _):
        hist_s[pl.ds(i * 16, 16)] = jnp.zeros((16,), jnp.float32)
        hist_d[pl.ds(i * 16, 16)] = jnp.zeros((16,), jnp.float32)
        return 0
    lax.fori_loop(0, HPAD // 16, zfill, 0)

    def body(i, _):
        iv = sidx_all[pl.ds(i * 16, 16)]
        cnt, last = plsc.scan_count(iv)
        plsc.addupdate_scatter(hist_s, [iv], cnt.astype(jnp.float32),
                               mask=last)
        iv2 = didx_all[pl.ds(i * 16, 16)]
        cnt2, last2 = plsc.scan_count(iv2)
        plsc.addupdate_scatter(hist_d, [iv2], cnt2.astype(jnp.float32),
                               mask=last2)
        return 0
    lax.fori_loop(0, EPW // 16, body, 0)

    for hist_ref, out_ref in ((hist_s, osrc_hbm), (hist_d, odst_hbm)):
        pltpu.sync_copy(hist_ref, stag.at[s])
        plsc.subcore_barrier()
        pltpu.sync_copy(stag.at[0, pl.ds(s * BPT, BPT)], acc)

        def rsum(r, _):
            pltpu.sync_copy(stag.at[r, pl.ds(s * BPT, BPT)], rbuf)

            def vadd(v, _):
                acc[pl.ds(v * 16, 16)] = (acc[pl.ds(v * 16, 16)]
                                          + rbuf[pl.ds(v * 16, 16)])
                return 0
            lax.fori_loop(0, BPT // 16, vadd, 0)
            return 0
        lax.fori_loop(1, NS, rsum, 0)
        pltpu.sync_copy(acc, out_ref.at[c, 0, pl.ds(s * BPT, BPT)])
        plsc.subcore_barrier()


@functools.partial(
    pl.kernel,
    out_type=jax.ShapeDtypeStruct((NC, NPAD, D), jnp.float32),
    mesh=_mesh,
    scratch_types=[
        pltpu.VMEM_SHARED((NPAD, D), jnp.float32),
        pltpu.VMEM((BLK, CH), jnp.int32),
        pltpu.VMEM((BLK, CH), jnp.int32),
        pltpu.VMEM((2, CH, D), jnp.float32),
        pltpu.SemaphoreType.DMA,
        pltpu.SemaphoreType.DMA,
    ],
)
def _agg_k(h_hbm, src2_hbm, dst2_hbm, out_hbm, agg_sh, sblk, dblk, rows2,
           sem0, sem1):
    c = lax.axis_index("c")
    s = lax.axis_index("s")
    w = c * NS + s

    def zrow(i, _):
        for j in range(D // 16):
            rows2[0, i, pl.ds(j * 16, 16)] = jnp.zeros((16,), jnp.float32)
        return 0
    lax.fori_loop(0, CH, zrow, 0)

    for q in range(4):
        pltpu.sync_copy(rows2.at[0], agg_sh.at[pl.ds(s * RPT + q * CH, CH)])
    pltpu.sync_copy(rows2.at[0, pl.ds(0, RPT - 4 * CH)],
                    agg_sh.at[pl.ds(s * RPT + 4 * CH, RPT - 4 * CH)])
    plsc.subcore_barrier()

    sems = (sem0, sem1)

    def gather(j, slot):
        pltpu.async_copy(h_hbm.at[sblk.at[j]], rows2.at[slot], sems[slot])

    def drain_scatter(j, slot):
        pltpu.make_async_copy(h_hbm.at[sblk.at[j]], rows2.at[slot],
                              sems[slot]).wait()
        pltpu.sync_copy(rows2.at[slot], agg_sh.at[dblk.at[j]], add=True)

    def blk_body(b, _):
        row0 = w * KCH + b * BLK
        pltpu.sync_copy(src2_hbm.at[pl.ds(row0, BLK), :], sblk)
        pltpu.sync_copy(dst2_hbm.at[pl.ds(row0, BLK), :], dblk)
        gather(0, 0)
        for j in range(0, BLK, 2):
            gather(j + 1, 1)
            drain_scatter(j, 0)
            if j + 2 < BLK:
                gather(j + 2, 0)
            drain_scatter(j + 1, 1)
        return 0
    lax.fori_loop(0, KCH // BLK, blk_body, 0)

    plsc.subcore_barrier()
    pltpu.sync_copy(agg_sh.at[pl.ds(s * RPT, RPT)],
                    out_hbm.at[c, pl.ds(s * RPT, RPT), :])


def _prep_body(hsrc_ref, feat_ref, w_ref, h_ref):
    p = jnp.dot(feat_ref[...], w_ref[...], preferred_element_type=jnp.float32)
    hs = hsrc_ref[0, 0, :] + hsrc_ref[1, 0, :]
    outd = jnp.maximum(hs[0:N], 1.0)
    nrm = lax.rsqrt(outd)
    h_ref[0:N, :] = p * nrm[:, None]
    h_ref[N:NPAD, :] = jnp.zeros((NPAD - N, D), jnp.float32)


def _final_body(agg_ref, hdst_ref, b_ref, out_ref):
    acc = agg_ref[0, 0:N, :] + agg_ref[1, 0:N, :]
    hd = hdst_ref[0, 0, :] + hdst_ref[1, 0, :]
    ind = jnp.maximum(hd[0:N], 1.0)
    out_ref[...] = acc * lax.rsqrt(ind)[:, None] + b_ref[0, :][None, :]


def kernel(feat, edge_index, W, b):
    pad = (jnp.arange(E2 - E, dtype=jnp.int32) % (NPAD - N)) + N
    srcp = jnp.concatenate([edge_index[0], pad])
    dstp = jnp.concatenate([edge_index[1], pad])

    hsrc, hdst = _hist_k(srcp, dstp)

    h = pl.pallas_call(
        _prep_body,
        out_shape=jax.ShapeDtypeStruct((NPAD, D), jnp.float32),
    )(hsrc, feat, W)

    agg = _agg_k(h, srcp.reshape(E2 // CH, CH),
                 dstp.reshape(E2 // CH, CH))

    out = pl.pallas_call(
        _final_body,
        out_shape=jax.ShapeDtypeStruct((N, D), jnp.float32),
    )(agg, hdst, b.reshape(1, D))
    return out


# speedup vs baseline: 1.1081x; 1.0403x over previous
import functools

import jax
import jax.numpy as jnp
from jax import lax
from jax.experimental import pallas as pl
from jax.experimental.pallas import tpu as pltpu
from jax.experimental.pallas import tpu_sc as plsc

N = 10000
D = 128
E = 320000

NC = 2
NS = 16
NW = NC * NS

CH = 128
EPW = 10240
E2 = NW * EPW
KCH = EPW // CH

NPAD = 10112
RPT = NPAD // NS
HPAD = 10240
BPT = HPAD // NS
BLK = 40

_mesh = plsc.VectorSubcoreMesh(
    core_axis_name="c", subcore_axis_name="s", num_cores=NC, num_subcores=NS)


@functools.partial(
    pl.kernel,
    out_type=(jax.ShapeDtypeStruct((NC, 1, HPAD), jnp.float32),
              jax.ShapeDtypeStruct((NC, 1, HPAD), jnp.float32)),
    mesh=_mesh,
    compiler_params=pltpu.CompilerParams(needs_layout_passes=False),
    scratch_types=[
        pltpu.VMEM_SHARED((NS, HPAD), jnp.float32),
        pltpu.VMEM((HPAD,), jnp.float32),
        pltpu.VMEM((HPAD,), jnp.float32),
        pltpu.VMEM((EPW,), jnp.int32),
        pltpu.VMEM((EPW,), jnp.int32),
        pltpu.VMEM((BPT,), jnp.float32),
        pltpu.VMEM((BPT,), jnp.float32),
    ],
)
def _hist_k(src_hbm, dst_hbm, osrc_hbm, odst_hbm, stag, hist_s, hist_d,
            sidx_all, didx_all, rbuf, acc):
    c = lax.axis_index("c")
    s = lax.axis_index("s")
    w = c * NS + s

    pltpu.sync_copy(src_hbm.at[pl.ds(w * EPW, EPW)], sidx_all)
    pltpu.sync_copy(dst_hbm.at[pl.ds(w * EPW, EPW)], didx_all)

    def zfill(i, _):
        hist_s[pl.ds(i * 16, 16)] = jnp.zeros((16,), jnp.float32)
        hist_d[pl.ds(i * 16, 16)] = jnp.zeros((16,), jnp.float32)
        return 0
    lax.fori_loop(0, HPAD // 16, zfill, 0)

    def body(i, _):
        iv = sidx_all[pl.ds(i * 16, 16)]
        cnt, last = plsc.scan_count(iv)
        plsc.addupdate_scatter(hist_s, [iv], cnt.astype(jnp.float32),
                               mask=last)
        iv2 = didx_all[pl.ds(i * 16, 16)]
        cnt2, last2 = plsc.scan_count(iv2)
        plsc.addupdate_scatter(hist_d, [iv2], cnt2.astype(jnp.float32),
                               mask=last2)
        return 0
    lax.fori_loop(0, EPW // 16, body, 0)

    for hist_ref, out_ref in ((hist_s, osrc_hbm), (hist_d, odst_hbm)):
        pltpu.sync_copy(hist_ref, stag.at[s])
        plsc.subcore_barrier()
        pltpu.sync_copy(stag.at[0, pl.ds(s * BPT, BPT)], acc)

        def rsum(r, _):
            pltpu.sync_copy(stag.at[r, pl.ds(s * BPT, BPT)], rbuf)

            def vadd(v, _):
                acc[pl.ds(v * 16, 16)] = (acc[pl.ds(v * 16, 16)]
                                          + rbuf[pl.ds(v * 16, 16)])
                return 0
            lax.fori_loop(0, BPT // 16, vadd, 0)
            return 0
        lax.fori_loop(1, NS, rsum, 0)
        pltpu.sync_copy(acc, out_ref.at[c, 0, pl.ds(s * BPT, BPT)])
        plsc.subcore_barrier()


@functools.partial(
    pl.kernel,
    out_type=jax.ShapeDtypeStruct((NC, NPAD, D), jnp.float32),
    mesh=_mesh,
    scratch_types=[
        pltpu.VMEM_SHARED((NPAD, D), jnp.float32),
        pltpu.VMEM((BLK, CH), jnp.int32),
        pltpu.VMEM((BLK, CH), jnp.int32),
        pltpu.VMEM((2, CH, D), jnp.float32),
        pltpu.SemaphoreType.DMA,
        pltpu.SemaphoreType.DMA,
    ],
)
def _agg_k(h_hbm, src2_hbm, dst2_hbm, out_hbm, agg_sh, sblk, dblk, rows2,
           sem0, sem1):
    c = lax.axis_index("c")
    s = lax.axis_index("s")
    w = c * NS + s

    def zrow(i, _):
        for j in range(D // 16):
            rows2[0, i, pl.ds(j * 16, 16)] = jnp.zeros((16,), jnp.float32)
        return 0
    lax.fori_loop(0, CH, zrow, 0)

    for q in range(4):
        pltpu.sync_copy(rows2.at[0], agg_sh.at[pl.ds(s * RPT + q * CH, CH)])
    pltpu.sync_copy(rows2.at[0, pl.ds(0, RPT - 4 * CH)],
                    agg_sh.at[pl.ds(s * RPT + 4 * CH, RPT - 4 * CH)])
    plsc.subcore_barrier()

    sems = (sem0, sem1)

    def gather(j, slot):
        pltpu.async_copy(h_hbm.at[sblk.at[j]], rows2.at[slot], sems[slot])

    def drain_scatter(j, slot):
        pltpu.make_async_copy(h_hbm.at[sblk.at[j]], rows2.at[slot],
                              sems[slot]).wait()
        pltpu.sync_copy(rows2.at[slot], agg_sh.at[dblk.at[j]], add=True)

    def blk_body(b, _):
        row0 = w * KCH + b * BLK
        pltpu.sync_copy(src2_hbm.at[pl.ds(row0, BLK), :], sblk)
        pltpu.sync_copy(dst2_hbm.at[pl.ds(row0, BLK), :], dblk)
        gather(0, 0)
        for j in range(0, BLK, 2):
            gather(j + 1, 1)
            drain_scatter(j, 0)
            if j + 2 < BLK:
                gather(j + 2, 0)
            drain_scatter(j + 1, 1)
        return 0
    lax.fori_loop(0, KCH // BLK, blk_body, 0)

    plsc.subcore_barrier()
    pltpu.sync_copy(agg_sh.at[pl.ds(s * RPT, RPT)],
                    out_hbm.at[c, pl.ds(s * RPT, RPT), :])


def _prep_body(hsrc_ref, feat_ref, w_ref, h_ref):
    p = jnp.dot(feat_ref[...], w_ref[...], preferred_element_type=jnp.float32)
    hs = hsrc_ref[0, 0, :] + hsrc_ref[1, 0, :]
    outd = jnp.maximum(hs[0:N], 1.0)
    nrm = lax.rsqrt(outd)
    h_ref[0:N, :] = p * nrm[:, None]
    h_ref[N:NPAD, :] = jnp.zeros((NPAD - N, D), jnp.float32)


def _final_body(agg_ref, hdst_ref, b_ref, out_ref):
    acc = agg_ref[0, 0:N, :] + agg_ref[1, 0:N, :]
    hd = hdst_ref[0, 0, :] + hdst_ref[1, 0, :]
    ind = jnp.maximum(hd[0:N], 1.0)
    out_ref[...] = acc * lax.rsqrt(ind)[:, None] + b_ref[0, :][None, :]


def kernel(feat, edge_index, W, b):
    pad = (jnp.arange(E2 - E, dtype=jnp.int32) % (NPAD - N)) + N
    srcp = jnp.concatenate([edge_index[0], pad])
    dstp = jnp.concatenate([edge_index[1], pad])

    hsrc, hdst = _hist_k(srcp, dstp)

    h = pl.pallas_call(
        _prep_body,
        out_shape=jax.ShapeDtypeStruct((NPAD, D), jnp.float32),
    )(hsrc, feat, W)

    agg = _agg_k(h, srcp.reshape(E2 // CH, CH),
                 dstp.reshape(E2 // CH, CH))

    out = pl.pallas_call(
        _final_body,
        out_shape=jax.ShapeDtypeStruct((N, D), jnp.float32),
    )(agg, hdst, b.reshape(1, D))
    return out
